# baseline (speedup 1.0000x reference)
import functools

import jax
import jax.numpy as jnp
from jax import lax
from jax.experimental import pallas as pl
from jax.experimental.pallas import tpu as pltpu
from jax.experimental.pallas import tpu_sc as plsc

N = 10000
E = 320000
IN_FEATS = 128
HEADS = 8
HID = 16
CLS = 40

NW = 32
EPW = E // NW
BLK = 112
NBLK = 90
SLOTS = NBLK * BLK
NPAD = 10112
RPS = NPAD // 16
NT = N + 16

A1W = 144
A2W = 48

_mesh = plsc.VectorSubcoreMesh(core_axis_name="c", subcore_axis_name="s")

_GDN = lax.GatherDimensionNumbers(
    offset_dims=(), collapsed_slice_dims=(0,), start_index_map=(0,))


def _bcast_lane(v, lane):
    idx = jnp.full((16, 1), lane, jnp.int32)
    return lax.gather(v, idx, _GDN, (1,),
                      mode=lax.GatherScatterMode.PROMISE_IN_BOUNDS)


def _dense1_body(x_ref, w_ref, alm_ref, arm_ref, a_ref, b_ref, c_ref):
    h = jnp.dot(x_ref[...], w_ref[...], preferred_element_type=jnp.float32)
    el = jnp.dot(h, alm_ref[...], preferred_element_type=jnp.float32)
    er = jnp.dot(h, arm_ref[...], preferred_element_type=jnp.float32)
    a_ref[0:N, 0:128] = h
    a_ref[0:N, 128:136] = el
    a_ref[0:N, 136:144] = jnp.zeros((N, 8), jnp.float32)
    a_ref[N:NT, 0:128] = jnp.zeros((16, 128), jnp.float32)
    a_ref[N:NT, 128:136] = jnp.full((16, 8), -1e30, jnp.float32)
    a_ref[N:NT, 136:144] = jnp.zeros((16, 8), jnp.float32)
    b_ref[:, 0:8] = er
    b_ref[:, 8:16] = jnp.zeros((N, 8), jnp.float32)
    cc = jnp.maximum(jnp.max(el, axis=0) + jnp.max(er, axis=0), 0.0)
    c_ref[0:1, 0:8] = cc.reshape(1, 8)
    c_ref[0:1, 8:16] = jnp.full((1, 8), 1e9, jnp.float32)


def _mid_body(p_ref, w2_ref, al2t_ref, ar2t_ref, rep_ref,
              a2_ref, el_ref, er_ref, c2_ref):
    acc = p_ref[0, 0:N] + p_ref[1, 0:N]
    num = acc[:, 0:128]
    den = jnp.maximum(acc[:, 128:136], 1e-9)
    dexp = jnp.dot(1.0 / den, rep_ref[...],
                   preferred_element_type=jnp.float32)
    out1 = num * dexp
    h2 = jnp.dot(out1, w2_ref[...], preferred_element_type=jnp.float32)
    el2 = jnp.dot(h2, al2t_ref[...], preferred_element_type=jnp.float32)
    er2 = jnp.dot(h2, ar2t_ref[...], preferred_element_type=jnp.float32)
    a2_ref[0:N, 0:40] = h2
    a2_ref[0:N, 40:48] = jnp.zeros((N, 8), jnp.float32)
    a2_ref[N:NT, :] = jnp.zeros((16, A2W), jnp.float32)
    el_ref[0:N, 0:1] = el2
    el_ref[N:NT, 0:1] = jnp.full((16, 1), -1e30, jnp.float32)
    er_ref[0:N, 0:1] = er2
    er_ref[N:NT, 0:1] = jnp.zeros((16, 1), jnp.float32)
    cc = jnp.maximum(jnp.max(el2) + jnp.max(er2), 0.0)
    c2_ref[...] = jnp.full((1, 16), 1.0, jnp.float32) * cc


def _final_body(p_ref, o_ref):
    acc = p_ref[0, 0:N] + p_ref[1, 0:N]
    den = jnp.maximum(acc[:, 40:41], 1e-9)
    o_ref[...] = acc[:, 0:40] / den


@functools.partial(
    pl.kernel,
    mesh=_mesh,
    compiler_params=pltpu.CompilerParams(use_tc_tiling_on_sc=False),
    out_type=jax.ShapeDtypeStruct((2, NPAD, A1W), jnp.float32),
    scratch_types=[
        pltpu.VMEM((BLK,), jnp.int32),
        pltpu.VMEM((BLK,), jnp.int32),
        pltpu.VMEM((BLK,), jnp.int32),
        pltpu.VMEM((BLK,), jnp.int32),
        pltpu.VMEM((BLK,), jnp.int32),
        pltpu.VMEM((BLK,), jnp.int32),
        pltpu.VMEM((BLK, A1W), jnp.float32),
        pltpu.VMEM((BLK, A1W), jnp.float32),
        pltpu.VMEM((BLK, 16), jnp.float32),
        pltpu.VMEM((BLK, 16), jnp.float32),
        pltpu.VMEM((16,), jnp.float32),
        pltpu.VMEM_SHARED((NPAD, A1W), jnp.float32),
        pltpu.SemaphoreType.DMA,
        pltpu.SemaphoreType.DMA,
        pltpu.SemaphoreType.DMA,
        pltpu.SemaphoreType.DMA,
        pltpu.SemaphoreType.DMA,
        pltpu.SemaphoreType.DMA,
        pltpu.SemaphoreType.DMA,
        pltpu.SemaphoreType.DMA,
        pltpu.SemaphoreType.DMA,
        pltpu.SemaphoreType.DMA,
    ],
)
def _edges1(a_hbm, b_hbm, src_hbm, dst_hbm, c_hbm, out_hbm,
            srcv0, srcv1, dstv0, dstv1, sdst0, sdst1,
            rows0, rows1, bv0, bv1, cv, acc,
            is0, is1, id0, id1, ga0, ga1, gb0, gb1, ss0, ss1):
    cid = lax.axis_index("c")
    sid = lax.axis_index("s")
    wid = sid * 2 + cid
    base0 = wid * SLOTS

    srcs = (srcv0, srcv1)
    dsts = (dstv0, dstv1)
    sdst = (sdst0, sdst1)
    rws = (rows0, rows1)
    bvs = (bv0, bv1)
    isS = (is0, is1)
    isD = (id0, id1)
    gA = (ga0, ga1)
    gB = (gb0, gb1)
    sS = (ss0, ss1)

    zz = jnp.zeros((16,), jnp.float32)

    @pl.loop(0, BLK)
    def _(r):
        for g in range(A1W // 16):
            rows0[r, pl.ds(16 * g, 16)] = zz

    for k in range(RPS // BLK):
        pltpu.sync_copy(rows0, acc.at[pl.ds(sid * RPS + k * BLK, BLK)])
    _rem = RPS - (RPS // BLK) * BLK
    pltpu.sync_copy(rows0.at[pl.ds(0, _rem)],
                    acc.at[pl.ds(sid * RPS + RPS - _rem, _rem)])
    pltpu.sync_copy(c_hbm, cv)
    plsc.subcore_barrier()
    ct = cv[...]

    def idx_start(b, p):
        pltpu.async_copy(src_hbm.at[pl.ds(base0 + b * BLK, BLK)],
                         srcs[p], isS[p])
        pltpu.async_copy(dst_hbm.at[pl.ds(base0 + b * BLK, BLK)],
                         dsts[p], isD[p])

    def idx_wait(p):
        pltpu.make_async_copy(src_hbm.at[pl.ds(base0, BLK)],
                              srcs[p], isS[p]).wait()
        pltpu.make_async_copy(dst_hbm.at[pl.ds(base0, BLK)],
                              dsts[p], isD[p]).wait()

    def g_start(p):
        pltpu.async_copy(a_hbm.at[srcs[p]], rws[p], gA[p])
        pltpu.async_copy(b_hbm.at[dsts[p]], bvs[p], gB[p])

    def g_wait(p):
        pltpu.make_async_copy(a_hbm.at[srcs[p]], rws[p], gA[p]).wait()
        pltpu.make_async_copy(b_hbm.at[dsts[p]], bvs[p], gB[p]).wait()

    def s_start(p):
        pltpu.async_copy(rws[p], acc.at[sdst[p]], sS[p], add=True)

    def s_wait(p):
        pltpu.make_async_copy(rws[p], acc.at[sdst[p]], sS[p]).wait()

    def compute(p):
        rw, bb = rws[p], bvs[p]

        @pl.loop(0, BLK, unroll=4)
        def _(i):
            rlast = rw[i, pl.ds(128, 16)]
            e = rlast + bb[i, :]
            e = jnp.where(e >= 0.0, e, 0.2 * e)
            ee = jnp.exp(e - ct)
            rw[i, pl.ds(128, 16)] = ee
            for h in range(8):
                bc = _bcast_lane(ee, h)
                rw[i, pl.ds(16 * h, 16)] = rw[i, pl.ds(16 * h, 16)] * bc

    pltpu.sync_copy(src_hbm.at[pl.ds(base0, BLK)], srcv0)
    pltpu.sync_copy(dst_hbm.at[pl.ds(base0, BLK)], dstv0)
    idx_start(1, 1)
    g_start(0)

    @pl.loop(0, NBLK, step=2)
    def _(b):
        for p in (0, 1):
            bb = b + p
            g_wait(p)

            @pl.loop(0, BLK, step=16)
            def _(j):
                sdst[p][pl.ds(j, 16)] = dsts[p][pl.ds(j, 16)]

            @pl.when(bb < NBLK - 2)
            def _():
                idx_start(bb + 2, p)

            @pl.when(bb >= 1)
            def _():
                s_wait(p ^ 1)

            @pl.when(bb < NBLK - 1)
            def _():
                idx_wait(p ^ 1)
                g_start(p ^ 1)

            compute(p)
            s_start(p)

    s_wait(1)
    plsc.subcore_barrier()
    pltpu.sync_copy(acc.at[pl.ds(sid * RPS, RPS)],
                    out_hbm.at[cid].at[pl.ds(sid * RPS, RPS)])


@functools.partial(
    pl.kernel,
    mesh=_mesh,
    compiler_params=pltpu.CompilerParams(use_tc_tiling_on_sc=False,
                                         needs_layout_passes=False),
    out_type=jax.ShapeDtypeStruct((2, NPAD, A2W), jnp.float32),
    scratch_types=[
        pltpu.VMEM((BLK,), jnp.int32),
        pltpu.VMEM((BLK,), jnp.int32),
        pltpu.VMEM((BLK,), jnp.int32),
        pltpu.VMEM((BLK,), jnp.int32),
        pltpu.VMEM((BLK,), jnp.int32),
        pltpu.VMEM((BLK,), jnp.int32),
        pltpu.VMEM((BLK, A2W), jnp.float32),
        pltpu.VMEM((BLK, A2W), jnp.float32),
        pltpu.VMEM((BLK, A2W), jnp.float32),
        pltpu.VMEM((BLK, A2W), jnp.float32),
        pltpu.VMEM((NT,), jnp.float32),
        pltpu.VMEM((NT,), jnp.float32),
        pltpu.VMEM((BLK,), jnp.float32),
        pltpu.VMEM((16,), jnp.float32),
        pltpu.VMEM_SHARED((NPAD, A2W), jnp.float32),
        pltpu.SemaphoreType.DMA,
        pltpu.SemaphoreType.DMA,
        pltpu.SemaphoreType.DMA,
        pltpu.SemaphoreType.DMA,
        pltpu.SemaphoreType.DMA,
        pltpu.SemaphoreType.DMA,
        pltpu.SemaphoreType.DMA,
        pltpu.SemaphoreType.DMA,
    ],
)
def _edges2(a_hbm, el_hbm, er_hbm, src_hbm, dst_hbm, c_hbm, out_hbm,
            srcv0, srcv1, dstv0, dstv1, sdst0, sdst1,
            rows0, rows1, msg0, msg1, elv, erv, eebuf, cv, acc,
            is0, is1, id0, id1, ga0, ga1, ss0, ss1):
    cid = lax.axis_index("c")
    sid = lax.axis_index("s")
    wid = sid * 2 + cid
    base0 = wid * SLOTS

    srcs = (srcv0, srcv1)
    dsts = (dstv0, dstv1)
    sdst = (sdst0, sdst1)
    rws = (rows0, rows1)
    msgs = (msg0, msg1)
    isS = (is0, is1)
    isD = (id0, id1)
    gA = (ga0, ga1)
    sS = (ss0, ss1)

    pltpu.sync_copy(el_hbm, elv)
    pltpu.sync_copy(er_hbm, erv)
    zz = jnp.zeros((16,), jnp.float32)

    @pl.loop(0, BLK)
    def _(r):
        for g in range(A2W // 16):
            rows0[r, pl.ds(16 * g, 16)] = zz

    for k in range(RPS // BLK):
        pltpu.sync_copy(rows0, acc.at[pl.ds(sid * RPS + k * BLK, BLK)])
    _rem = RPS - (RPS // BLK) * BLK
    pltpu.sync_copy(rows0.at[pl.ds(0, _rem)],
                    acc.at[pl.ds(sid * RPS + RPS - _rem, _rem)])
    pltpu.sync_copy(c_hbm, cv)
    plsc.subcore_barrier()
    ct = cv[...]
    eq8 = lax.iota(jnp.int32, 16) == 8

    def idx_start(b, p):
        pltpu.async_copy(src_hbm.at[pl.ds(base0 + b * BLK, BLK)],
                         srcs[p], isS[p])
        pltpu.async_copy(dst_hbm.at[pl.ds(base0 + b * BLK, BLK)],
                         dsts[p], isD[p])

    def idx_wait(p):
        pltpu.make_async_copy(src_hbm.at[pl.ds(base0, BLK)],
                              srcs[p], isS[p]).wait()
        pltpu.make_async_copy(dst_hbm.at[pl.ds(base0, BLK)],
                              dsts[p], isD[p]).wait()

    def g_start(p):
        pltpu.async_copy(a_hbm.at[srcs[p]], rws[p], gA[p])

    def g_wait(p):
        pltpu.make_async_copy(a_hbm.at[srcs[p]], rws[p], gA[p]).wait()

    def s_start(p):
        pltpu.async_copy(msgs[p], acc.at[sdst[p]], sS[p], add=True)

    def s_wait(p):
        pltpu.make_async_copy(msgs[p], acc.at[sdst[p]], sS[p]).wait()

    def compute_ee(p):
        sv, dv = srcs[p], dsts[p]

        @pl.loop(0, BLK, step=16)
        def _(j):
            s16 = sv[pl.ds(j, 16)]
            d16 = dv[pl.ds(j, 16)]
            e = plsc.load_gather(elv, [s16]) + plsc.load_gather(erv, [d16])
            e = jnp.where(e >= 0.0, e, 0.2 * e)
            eebuf[pl.ds(j, 16)] = jnp.exp(e - ct)

    def compute_mul(p):
        rw, mg = rws[p], msgs[p]

        @pl.loop(0, BLK, step=16)
        def _(j):
            ee16 = eebuf[pl.ds(j, 16)]
            for k in range(16):
                i = j + k
                bc = _bcast_lane(ee16, k)
                eei = jnp.where(eq8, bc, 0.0)
                mg[i, pl.ds(0, 16)] = rw[i, pl.ds(0, 16)] * bc
                mg[i, pl.ds(16, 16)] = rw[i, pl.ds(16, 16)] * bc
                mg[i, pl.ds(32, 16)] = rw[i, pl.ds(32, 16)] * bc + eei

    pltpu.sync_copy(src_hbm.at[pl.ds(base0, BLK)], srcv0)
    pltpu.sync_copy(dst_hbm.at[pl.ds(base0, BLK)], dstv0)
    idx_start(1, 1)
    g_start(0)

    @pl.loop(0, NBLK, step=2)
    def _(b):
        for p in (0, 1):
            bb = b + p
            g_wait(p)

            @pl.loop(0, BLK, step=16)
            def _(j):
                sdst[p][pl.ds(j, 16)] = dsts[p][pl.ds(j, 16)]

            compute_ee(p)

            @pl.when(bb < NBLK - 2)
            def _():
                idx_start(bb + 2, p)

            @pl.when(bb < NBLK - 1)
            def _():
                idx_wait(p ^ 1)
                g_start(p ^ 1)

            @pl.when(bb >= 2)
            def _():
                s_wait(p)

            compute_mul(p)
            s_start(p)

    s_wait(0)
    s_wait(1)
    plsc.subcore_barrier()
    pltpu.sync_copy(acc.at[pl.ds(sid * RPS, RPS)],
                    out_hbm.at[cid].at[pl.ds(sid * RPS, RPS)])


def kernel(features, edge_index, W1, al1, ar1, W2, al2, ar2):
    src = edge_index[0].astype(jnp.int32)
    dst = edge_index[1].astype(jnp.int32)
    pad = SLOTS - EPW
    srcp = jnp.pad(src.reshape(NW, EPW), ((0, 0), (0, pad)),
                   constant_values=N).reshape(-1)
    dstp = jnp.pad(dst.reshape(NW, EPW), ((0, 0), (0, pad)),
                   constant_values=0).reshape(-1)

    eye8 = jnp.eye(HEADS, dtype=jnp.float32)
    alm = (al1[:, :, None] * eye8[:, None, :]).reshape(HEADS * HID, HEADS)
    arm = (ar1[:, :, None] * eye8[:, None, :]).reshape(HEADS * HID, HEADS)
    rep = jnp.repeat(eye8, HID, axis=1)

    a1, b1, c1 = pl.pallas_call(
        _dense1_body,
        out_shape=[
            jax.ShapeDtypeStruct((NT, A1W), jnp.float32),
            jax.ShapeDtypeStruct((N, 16), jnp.float32),
            jax.ShapeDtypeStruct((1, 16), jnp.float32),
        ],
    )(features, W1, alm, arm)

    p1 = _edges1(a1, b1, srcp, dstp, c1.reshape(16))

    a2, elv, erv, c2 = pl.pallas_call(
        _mid_body,
        out_shape=[
            jax.ShapeDtypeStruct((NT, A2W), jnp.float32),
            jax.ShapeDtypeStruct((NT, 1), jnp.float32),
            jax.ShapeDtypeStruct((NT, 1), jnp.float32),
            jax.ShapeDtypeStruct((1, 16), jnp.float32),
        ],
    )(p1, W2, al2.T, ar2.T, rep)

    p2 = _edges2(a2, elv.reshape(NT), erv.reshape(NT), srcp, dstp,
                 c2.reshape(16))

    out = pl.pallas_call(
        _final_body,
        out_shape=jax.ShapeDtypeStruct((N, CLS), jnp.float32),
    )(p2)
    return out


# speedup vs baseline: 1.0085x; 1.0085x over previous
import functools

import jax
import jax.numpy as jnp
from jax import lax
from jax.experimental import pallas as pl
from jax.experimental.pallas import tpu as pltpu
from jax.experimental.pallas import tpu_sc as plsc

N = 10000
E = 320000
IN_FEATS = 128
HEADS = 8
HID = 16
CLS = 40

NW = 32
EPW = E // NW
BLK = 112
NBLK = 90
SLOTS = NBLK * BLK
NPAD = 10112
RPS = NPAD // 16
NT = N + 16

A1W = 144
A2W = 48

_mesh = plsc.VectorSubcoreMesh(core_axis_name="c", subcore_axis_name="s")

_GDN = lax.GatherDimensionNumbers(
    offset_dims=(), collapsed_slice_dims=(0,), start_index_map=(0,))


def _bcast_lane(v, lane):
    idx = jnp.full((16, 1), lane, jnp.int32)
    return lax.gather(v, idx, _GDN, (1,),
                      mode=lax.GatherScatterMode.PROMISE_IN_BOUNDS)


def _dense1_body(x_ref, w_ref, alm_ref, arm_ref, a_ref, b_ref, c_ref):
    h = jnp.dot(x_ref[...], w_ref[...], preferred_element_type=jnp.float32)
    el = jnp.dot(h, alm_ref[...], preferred_element_type=jnp.float32)
    er = jnp.dot(h, arm_ref[...], preferred_element_type=jnp.float32)
    a_ref[0:N, 0:128] = h
    a_ref[0:N, 128:136] = el
    a_ref[0:N, 136:144] = jnp.zeros((N, 8), jnp.float32)
    a_ref[N:NT, 0:128] = jnp.zeros((16, 128), jnp.float32)
    a_ref[N:NT, 128:136] = jnp.full((16, 8), -1e30, jnp.float32)
    a_ref[N:NT, 136:144] = jnp.zeros((16, 8), jnp.float32)
    b_ref[:, 0:8] = er
    b_ref[:, 8:16] = jnp.zeros((N, 8), jnp.float32)
    cc = jnp.maximum(jnp.max(el, axis=0) + jnp.max(er, axis=0), 0.0)
    c_ref[0:1, 0:8] = cc.reshape(1, 8)
    c_ref[0:1, 8:16] = jnp.full((1, 8), 1e9, jnp.float32)


def _mid_body(p_ref, w2_ref, al2t_ref, ar2t_ref, rep_ref,
              a2_ref, el_ref, er_ref, c2_ref):
    acc = p_ref[0, 0:N] + p_ref[1, 0:N]
    num = acc[:, 0:128]
    den = jnp.maximum(acc[:, 128:136], 1e-9)
    dexp = jnp.dot(1.0 / den, rep_ref[...],
                   preferred_element_type=jnp.float32)
    out1 = num * dexp
    h2 = jnp.dot(out1, w2_ref[...], preferred_element_type=jnp.float32)
    el2 = jnp.dot(h2, al2t_ref[...], preferred_element_type=jnp.float32)
    er2 = jnp.dot(h2, ar2t_ref[...], preferred_element_type=jnp.float32)
    a2_ref[0:N, 0:40] = h2
    a2_ref[0:N, 40:48] = jnp.zeros((N, 8), jnp.float32)
    a2_ref[N:NT, :] = jnp.zeros((16, A2W), jnp.float32)
    el_ref[0:N, 0:1] = el2
    el_ref[N:NT, 0:1] = jnp.full((16, 1), -1e30, jnp.float32)
    er_ref[0:N, 0:1] = er2
    er_ref[N:NT, 0:1] = jnp.zeros((16, 1), jnp.float32)
    cc = jnp.maximum(jnp.max(el2) + jnp.max(er2), 0.0)
    c2_ref[...] = jnp.full((1, 16), 1.0, jnp.float32) * cc


def _final_body(p_ref, o_ref):
    acc = p_ref[0, 0:N] + p_ref[1, 0:N]
    den = jnp.maximum(acc[:, 40:41], 1e-9)
    o_ref[...] = acc[:, 0:40] / den


@functools.partial(
    pl.kernel,
    mesh=_mesh,
    compiler_params=pltpu.CompilerParams(use_tc_tiling_on_sc=False),
    out_type=jax.ShapeDtypeStruct((2, NPAD, A1W), jnp.float32),
    scratch_types=[
        pltpu.VMEM((BLK,), jnp.int32),
        pltpu.VMEM((BLK,), jnp.int32),
        pltpu.VMEM((BLK,), jnp.int32),
        pltpu.VMEM((BLK,), jnp.int32),
        pltpu.VMEM((BLK,), jnp.int32),
        pltpu.VMEM((BLK,), jnp.int32),
        pltpu.VMEM((BLK, A1W), jnp.float32),
        pltpu.VMEM((BLK, A1W), jnp.float32),
        pltpu.VMEM((BLK, 16), jnp.float32),
        pltpu.VMEM((BLK, 16), jnp.float32),
        pltpu.VMEM((16,), jnp.float32),
        pltpu.VMEM_SHARED((NPAD, A1W), jnp.float32),
        pltpu.SemaphoreType.DMA,
        pltpu.SemaphoreType.DMA,
        pltpu.SemaphoreType.DMA,
        pltpu.SemaphoreType.DMA,
        pltpu.SemaphoreType.DMA,
        pltpu.SemaphoreType.DMA,
        pltpu.SemaphoreType.DMA,
        pltpu.SemaphoreType.DMA,
        pltpu.SemaphoreType.DMA,
        pltpu.SemaphoreType.DMA,
    ],
)
def _edges1(a_hbm, b_hbm, src_hbm, dst_hbm, c_hbm, out_hbm,
            srcv0, srcv1, dstv0, dstv1, sdst0, sdst1,
            rows0, rows1, bv0, bv1, cv, acc,
            is0, is1, id0, id1, ga0, ga1, gb0, gb1, ss0, ss1):
    cid = lax.axis_index("c")
    sid = lax.axis_index("s")
    wid = sid * 2 + cid
    base0 = wid * SLOTS

    srcs = (srcv0, srcv1)
    dsts = (dstv0, dstv1)
    sdst = (sdst0, sdst1)
    rws = (rows0, rows1)
    bvs = (bv0, bv1)
    isS = (is0, is1)
    isD = (id0, id1)
    gA = (ga0, ga1)
    gB = (gb0, gb1)
    sS = (ss0, ss1)

    zz = jnp.zeros((16,), jnp.float32)

    @pl.loop(0, BLK)
    def _(r):
        for g in range(A1W // 16):
            rows0[r, pl.ds(16 * g, 16)] = zz

    for k in range(RPS // BLK):
        pltpu.sync_copy(rows0, acc.at[pl.ds(sid * RPS + k * BLK, BLK)])
    _rem = RPS - (RPS // BLK) * BLK
    pltpu.sync_copy(rows0.at[pl.ds(0, _rem)],
                    acc.at[pl.ds(sid * RPS + RPS - _rem, _rem)])
    pltpu.sync_copy(c_hbm, cv)
    plsc.subcore_barrier()
    ct = cv[...]

    def idx_start(b, p):
        pltpu.async_copy(src_hbm.at[pl.ds(base0 + b * BLK, BLK)],
                         srcs[p], isS[p])
        pltpu.async_copy(dst_hbm.at[pl.ds(base0 + b * BLK, BLK)],
                         dsts[p], isD[p])

    def idx_wait(p):
        pltpu.make_async_copy(src_hbm.at[pl.ds(base0, BLK)],
                              srcs[p], isS[p]).wait()
        pltpu.make_async_copy(dst_hbm.at[pl.ds(base0, BLK)],
                              dsts[p], isD[p]).wait()

    def g_start(p):
        pltpu.async_copy(a_hbm.at[srcs[p]], rws[p], gA[p])
        pltpu.async_copy(b_hbm.at[dsts[p]], bvs[p], gB[p])

    def g_wait(p):
        pltpu.make_async_copy(a_hbm.at[srcs[p]], rws[p], gA[p]).wait()
        pltpu.make_async_copy(b_hbm.at[dsts[p]], bvs[p], gB[p]).wait()

    def s_start(p):
        pltpu.async_copy(rws[p], acc.at[sdst[p]], sS[p], add=True)

    def s_wait(p):
        pltpu.make_async_copy(rws[p], acc.at[sdst[p]], sS[p]).wait()

    def compute(p):
        rw, bb = rws[p], bvs[p]

        @pl.loop(0, BLK, unroll=2)
        def _(i):
            rlast = rw[i, pl.ds(128, 16)]
            e = rlast + bb[i, :]
            e = jnp.where(e >= 0.0, e, 0.2 * e)
            ee = jnp.exp(e - ct)
            rw[i, pl.ds(128, 16)] = ee
            for h in range(8):
                bc = _bcast_lane(ee, h)
                rw[i, pl.ds(16 * h, 16)] = rw[i, pl.ds(16 * h, 16)] * bc

    pltpu.sync_copy(src_hbm.at[pl.ds(base0, BLK)], srcv0)
    pltpu.sync_copy(dst_hbm.at[pl.ds(base0, BLK)], dstv0)
    idx_start(1, 1)
    g_start(0)

    @pl.loop(0, NBLK, step=2)
    def _(b):
        for p in (0, 1):
            bb = b + p
            g_wait(p)

            @pl.loop(0, BLK, step=16)
            def _(j):
                sdst[p][pl.ds(j, 16)] = dsts[p][pl.ds(j, 16)]

            @pl.when(bb < NBLK - 2)
            def _():
                idx_start(bb + 2, p)

            @pl.when(bb >= 1)
            def _():
                s_wait(p ^ 1)

            @pl.when(bb < NBLK - 1)
            def _():
                idx_wait(p ^ 1)
                g_start(p ^ 1)

            compute(p)
            s_start(p)

    s_wait(1)
    plsc.subcore_barrier()
    pltpu.sync_copy(acc.at[pl.ds(sid * RPS, RPS)],
                    out_hbm.at[cid].at[pl.ds(sid * RPS, RPS)])


@functools.partial(
    pl.kernel,
    mesh=_mesh,
    compiler_params=pltpu.CompilerParams(use_tc_tiling_on_sc=False,
                                         needs_layout_passes=False),
    out_type=jax.ShapeDtypeStruct((2, NPAD, A2W), jnp.float32),
    scratch_types=[
        pltpu.VMEM((BLK,), jnp.int32),
        pltpu.VMEM((BLK,), jnp.int32),
        pltpu.VMEM((BLK,), jnp.int32),
        pltpu.VMEM((BLK,), jnp.int32),
        pltpu.VMEM((BLK,), jnp.int32),
        pltpu.VMEM((BLK,), jnp.int32),
        pltpu.VMEM((BLK, A2W), jnp.float32),
        pltpu.VMEM((BLK, A2W), jnp.float32),
        pltpu.VMEM((BLK, A2W), jnp.float32),
        pltpu.VMEM((BLK, A2W), jnp.float32),
        pltpu.VMEM((NT,), jnp.float32),
        pltpu.VMEM((NT,), jnp.float32),
        pltpu.VMEM((BLK,), jnp.float32),
        pltpu.VMEM((16,), jnp.float32),
        pltpu.VMEM_SHARED((NPAD, A2W), jnp.float32),
        pltpu.SemaphoreType.DMA,
        pltpu.SemaphoreType.DMA,
        pltpu.SemaphoreType.DMA,
        pltpu.SemaphoreType.DMA,
        pltpu.SemaphoreType.DMA,
        pltpu.SemaphoreType.DMA,
        pltpu.SemaphoreType.DMA,
        pltpu.SemaphoreType.DMA,
    ],
)
def _edges2(a_hbm, el_hbm, er_hbm, src_hbm, dst_hbm, c_hbm, out_hbm,
            srcv0, srcv1, dstv0, dstv1, sdst0, sdst1,
            rows0, rows1, msg0, msg1, elv, erv, eebuf, cv, acc,
            is0, is1, id0, id1, ga0, ga1, ss0, ss1):
    cid = lax.axis_index("c")
    sid = lax.axis_index("s")
    wid = sid * 2 + cid
    base0 = wid * SLOTS

    srcs = (srcv0, srcv1)
    dsts = (dstv0, dstv1)
    sdst = (sdst0, sdst1)
    rws = (rows0, rows1)
    msgs = (msg0, msg1)
    isS = (is0, is1)
    isD = (id0, id1)
    gA = (ga0, ga1)
    sS = (ss0, ss1)

    pltpu.sync_copy(el_hbm, elv)
    pltpu.sync_copy(er_hbm, erv)
    zz = jnp.zeros((16,), jnp.float32)

    @pl.loop(0, BLK)
    def _(r):
        for g in range(A2W // 16):
            rows0[r, pl.ds(16 * g, 16)] = zz

    for k in range(RPS // BLK):
        pltpu.sync_copy(rows0, acc.at[pl.ds(sid * RPS + k * BLK, BLK)])
    _rem = RPS - (RPS // BLK) * BLK
    pltpu.sync_copy(rows0.at[pl.ds(0, _rem)],
                    acc.at[pl.ds(sid * RPS + RPS - _rem, _rem)])
    pltpu.sync_copy(c_hbm, cv)
    plsc.subcore_barrier()
    ct = cv[...]
    eq8 = lax.iota(jnp.int32, 16) == 8

    def idx_start(b, p):
        pltpu.async_copy(src_hbm.at[pl.ds(base0 + b * BLK, BLK)],
                         srcs[p], isS[p])
        pltpu.async_copy(dst_hbm.at[pl.ds(base0 + b * BLK, BLK)],
                         dsts[p], isD[p])

    def idx_wait(p):
        pltpu.make_async_copy(src_hbm.at[pl.ds(base0, BLK)],
                              srcs[p], isS[p]).wait()
        pltpu.make_async_copy(dst_hbm.at[pl.ds(base0, BLK)],
                              dsts[p], isD[p]).wait()

    def g_start(p):
        pltpu.async_copy(a_hbm.at[srcs[p]], rws[p], gA[p])

    def g_wait(p):
        pltpu.make_async_copy(a_hbm.at[srcs[p]], rws[p], gA[p]).wait()

    def s_start(p):
        pltpu.async_copy(msgs[p], acc.at[sdst[p]], sS[p], add=True)

    def s_wait(p):
        pltpu.make_async_copy(msgs[p], acc.at[sdst[p]], sS[p]).wait()

    def compute_ee(p):
        sv, dv = srcs[p], dsts[p]

        @pl.loop(0, BLK, step=16)
        def _(j):
            s16 = sv[pl.ds(j, 16)]
            d16 = dv[pl.ds(j, 16)]
            e = plsc.load_gather(elv, [s16]) + plsc.load_gather(erv, [d16])
            e = jnp.where(e >= 0.0, e, 0.2 * e)
            eebuf[pl.ds(j, 16)] = jnp.exp(e - ct)

    def compute_mul(p):
        rw, mg = rws[p], msgs[p]

        @pl.loop(0, BLK, step=16)
        def _(j):
            ee16 = eebuf[pl.ds(j, 16)]
            for k in range(16):
                i = j + k
                bc = _bcast_lane(ee16, k)
                eei = jnp.where(eq8, bc, 0.0)
                mg[i, pl.ds(0, 16)] = rw[i, pl.ds(0, 16)] * bc
                mg[i, pl.ds(16, 16)] = rw[i, pl.ds(16, 16)] * bc
                mg[i, pl.ds(32, 16)] = rw[i, pl.ds(32, 16)] * bc + eei

    pltpu.sync_copy(src_hbm.at[pl.ds(base0, BLK)], srcv0)
    pltpu.sync_copy(dst_hbm.at[pl.ds(base0, BLK)], dstv0)
    idx_start(1, 1)
    g_start(0)

    @pl.loop(0, NBLK, step=2)
    def _(b):
        for p in (0, 1):
            bb = b + p
            g_wait(p)

            @pl.loop(0, BLK, step=16)
            def _(j):
                sdst[p][pl.ds(j, 16)] = dsts[p][pl.ds(j, 16)]

            compute_ee(p)

            @pl.when(bb < NBLK - 2)
            def _():
                idx_start(bb + 2, p)

            @pl.when(bb < NBLK - 1)
            def _():
                idx_wait(p ^ 1)
                g_start(p ^ 1)

            @pl.when(bb >= 2)
            def _():
                s_wait(p)

            compute_mul(p)
            s_start(p)

    s_wait(0)
    s_wait(1)
    plsc.subcore_barrier()
    pltpu.sync_copy(acc.at[pl.ds(sid * RPS, RPS)],
                    out_hbm.at[cid].at[pl.ds(sid * RPS, RPS)])


def kernel(features, edge_index, W1, al1, ar1, W2, al2, ar2):
    src = edge_index[0].astype(jnp.int32)
    dst = edge_index[1].astype(jnp.int32)
    pad = SLOTS - EPW
    srcp = jnp.pad(src.reshape(NW, EPW), ((0, 0), (0, pad)),
                   constant_values=N).reshape(-1)
    dstp = jnp.pad(dst.reshape(NW, EPW), ((0, 0), (0, pad)),
                   constant_values=0).reshape(-1)

    eye8 = jnp.eye(HEADS, dtype=jnp.float32)
    alm = (al1[:, :, None] * eye8[:, None, :]).reshape(HEADS * HID, HEADS)
    arm = (ar1[:, :, None] * eye8[:, None, :]).reshape(HEADS * HID, HEADS)
    rep = jnp.repeat(eye8, HID, axis=1)

    a1, b1, c1 = pl.pallas_call(
        _dense1_body,
        out_shape=[
            jax.ShapeDtypeStruct((NT, A1W), jnp.float32),
            jax.ShapeDtypeStruct((N, 16), jnp.float32),
            jax.ShapeDtypeStruct((1, 16), jnp.float32),
        ],
    )(features, W1, alm, arm)

    p1 = _edges1(a1, b1, srcp, dstp, c1.reshape(16))

    a2, elv, erv, c2 = pl.pallas_call(
        _mid_body,
        out_shape=[
            jax.ShapeDtypeStruct((NT, A2W), jnp.float32),
            jax.ShapeDtypeStruct((NT, 1), jnp.float32),
            jax.ShapeDtypeStruct((NT, 1), jnp.float32),
            jax.ShapeDtypeStruct((1, 16), jnp.float32),
        ],
    )(p1, W2, al2.T, ar2.T, rep)

    p2 = _edges2(a2, elv.reshape(NT), erv.reshape(NT), srcp, dstp,
                 c2.reshape(16))

    out = pl.pallas_call(
        _final_body,
        out_shape=jax.ShapeDtypeStruct((N, CLS), jnp.float32),
    )(p2)
    return out
